# baseline (speedup 1.0000x reference)
import functools
import jax
import jax.numpy as jnp
from jax import lax
from jax.experimental import pallas as pl
from jax.experimental.pallas import tpu as pltpu
from jax.experimental.pallas import tpu_sc as plsc

N = 100000
DIM = 32
HALF = 50000
CH = 128
ROWS = 12544
E_PAD = ROWS * CH
SENT = 100000
HPAD = 100352
HSL = HPAD // 16
HW = 16
HSW = 392
AK = 56
AG = 7
APAD = 50048
TRASH = HALF
NWCH = APAD // CH

A_ROWS = ROWS // 32
C_ROWS = ROWS // 16
CK = 28
CG = C_ROWS // CK
NB = 4
ST = CK // NB

RB = 6272

_mesh = plsc.VectorSubcoreMesh(core_axis_name="c", subcore_axis_name="s")


@functools.partial(
    pl.kernel,
    out_type=jax.ShapeDtypeStruct((2, HPAD, HW), jnp.float32),
    mesh=_mesh,
    compiler_params=pltpu.CompilerParams(use_tc_tiling_on_sc=False),
    scratch_types=[
        pltpu.VMEM((AK, CH), jnp.int32),
        pltpu.VMEM((AK, CH), jnp.int32),
        pltpu.VMEM((CH, HW), jnp.float32),
        pltpu.VMEM((CH, HW), jnp.float32),
        pltpu.VMEM((HSW, HW), jnp.float32),
        pltpu.VMEM_SHARED((HPAD, HW), jnp.float32),
    ],
)
def _deg_kernel(src_hbm, dst_hbm, onl_hbm, onr_hbm, zcol_hbm, deg_hbm,
                srcb, dstb, onlb, onrb, slabb, hist):
    c = lax.axis_index("c")
    s = lax.axis_index("s")
    w = s * 2 + c

    pltpu.sync_copy(zcol_hbm, slabb)
    for k in range(16):
        pltpu.sync_copy(slabb, hist.at[pl.ds(s * HSL + k * HSW, HSW)])
    pltpu.sync_copy(onl_hbm, onlb)
    pltpu.sync_copy(onr_hbm, onrb)
    plsc.subcore_barrier()

    def group(g, carry):
        r0 = w * A_ROWS + g * AK
        pltpu.sync_copy(src_hbm.at[pl.ds(r0, AK)], srcb)
        pltpu.sync_copy(dst_hbm.at[pl.ds(r0, AK)], dstb)

        def body(j, cc):
            pltpu.sync_copy(onlb, hist.at[srcb.at[j]], add=True)
            pltpu.sync_copy(onrb, hist.at[dstb.at[j]], add=True)
            return cc

        lax.fori_loop(0, AK, body, 0)
        return carry

    lax.fori_loop(0, AG, group, 0)
    plsc.subcore_barrier()

    for k in range(16):
        r = s * HSL + k * HSW
        pltpu.sync_copy(hist.at[pl.ds(r, HSW)], slabb)
        pltpu.sync_copy(slabb, deg_hbm.at[c, pl.ds(r, HSW)])


@functools.partial(
    pl.kernel,
    out_type=jax.ShapeDtypeStruct((N, DIM), jnp.float32),
    mesh=_mesh,
    compiler_params=pltpu.CompilerParams(use_tc_tiling_on_sc=False),
    scratch_types=(
        [pltpu.VMEM((CK, CH), jnp.int32)] * 2
        + [pltpu.VMEM((CH, DIM), jnp.float32)] * NB
        + [pltpu.SemaphoreType.DMA] * (2 * NB)
        + [pltpu.VMEM_SHARED((APAD, DIM), jnp.float32)]
    ),
)
def _agg_kernel(h_hbm, src_hbm, dst_hbm, zrow_hbm, out_hbm,
                srcb, dstb, r0b, r1b, r2b, r3b,
                g0, g1, g2, g3, s0, s1, s2, s3, acc):
    c = lax.axis_index("c")
    s = lax.axis_index("s")
    base = c * HALF
    rows = [r0b, r1b, r2b, r3b]
    gsem = [g0, g1, g2, g3]
    ssem = [s0, s1, s2, s3]

    pltpu.sync_copy(zrow_hbm, r0b)
    for k in range(25):
        ch = s + 16 * k

        @pl.when(ch < NWCH)
        def _():
            pltpu.sync_copy(r0b, acc.at[pl.ds(ch * CH, CH)])

    plsc.subcore_barrier()

    def group(g, carry):
        @pl.when(g > 0)
        def _():
            for b in range(NB):
                pltpu.make_async_copy(
                    rows[b], acc.at[dstb.at[0]], ssem[b]).wait()

        r0 = s * C_ROWS + g * CK
        pltpu.sync_copy(src_hbm.at[pl.ds(r0, CK)], srcb)
        pltpu.sync_copy(dst_hbm.at[pl.ds(r0, CK)], dstb)

        def crow(j, cc):
            def cvec(i, ci):
                d = dstb[j, pl.ds(i * 16, 16)]
                loc = d - base
                ok = (loc >= 0) & (loc < HALF)
                dstb[j, pl.ds(i * 16, 16)] = jnp.where(ok, loc, TRASH)
                return ci
            return lax.fori_loop(0, CH // 16, cvec, cc)

        lax.fori_loop(0, CK, crow, 0)

        def step(st, cc):
            descs = []
            for b in range(NB):
                @pl.when(st > 0)
                def _():
                    pltpu.make_async_copy(
                        rows[b], acc.at[dstb.at[0]], ssem[b]).wait()
                descs.append(pltpu.async_copy(
                    h_hbm.at[srcb.at[st * NB + b]], rows[b], gsem[b]))
            for b in range(NB):
                descs[b].wait()
                pltpu.async_copy(rows[b], acc.at[dstb.at[st * NB + b]],
                                 ssem[b], add=True)
            return cc

        lax.fori_loop(0, ST, step, 0)
        return carry

    lax.fori_loop(0, CG, group, 0)

    for b in range(NB):
        pltpu.make_async_copy(rows[b], acc.at[dstb.at[0]], ssem[b]).wait()
    plsc.subcore_barrier()

    for k in range(25):
        ch = s + 16 * k

        @pl.when(ch < NWCH - 1)
        def _():
            pltpu.sync_copy(acc.at[pl.ds(ch * CH, CH)], r0b)
            pltpu.sync_copy(r0b, out_hbm.at[pl.ds(base + ch * CH, CH)])

        @pl.when(ch == NWCH - 1)
        def _():
            pltpu.sync_copy(acc.at[pl.ds(ch * CH, 80)],
                            r0b.at[pl.ds(0, 80)])
            pltpu.sync_copy(r0b.at[pl.ds(0, 80)],
                            out_hbm.at[pl.ds(base + ch * CH, 80)])


def _norm_body(x_ref, deg_ref, h_ref, nd_ref):
    od = deg_ref[0, :, 0:1] + deg_ref[1, :, 0:1]
    idg = deg_ref[0, :, 8:9] + deg_ref[1, :, 8:9]
    ns = jnp.where(od > 0, lax.rsqrt(jnp.maximum(od, 1.0)), 0.0)
    nd = jnp.where(idg > 0, lax.rsqrt(jnp.maximum(idg, 1.0)), 0.0)
    h_ref[...] = x_ref[...] * ns
    nd_ref[...] = nd


_norm_call = pl.pallas_call(
    _norm_body,
    grid=(16,),
    in_specs=[
        pl.BlockSpec((RB, DIM), lambda i: (i, 0)),
        pl.BlockSpec((2, RB, HW), lambda i: (0, i, 0)),
    ],
    out_specs=(
        pl.BlockSpec((RB, DIM), lambda i: (i, 0)),
        pl.BlockSpec((RB, 1), lambda i: (i, 0)),
    ),
    out_shape=(
        jax.ShapeDtypeStruct((HPAD, DIM), jnp.float32),
        jax.ShapeDtypeStruct((HPAD, 1), jnp.float32),
    ),
)


def _proj_body(agg_ref, nd_ref, w_ref, b_ref, o_ref):
    a = agg_ref[...] * nd_ref[...]
    o_ref[...] = jnp.dot(a, w_ref[...],
                         preferred_element_type=jnp.float32) + b_ref[...]


_proj_call = pl.pallas_call(
    _proj_body,
    grid=(16,),
    in_specs=[
        pl.BlockSpec((RB, DIM), lambda i: (i, 0)),
        pl.BlockSpec((RB, 1), lambda i: (i, 0)),
        pl.BlockSpec((DIM, DIM), lambda i: (0, 0)),
        pl.BlockSpec((1, DIM), lambda i: (0, 0)),
    ],
    out_specs=pl.BlockSpec((RB, DIM), lambda i: (i, 0)),
    out_shape=jax.ShapeDtypeStruct((N, DIM), jnp.float32),
)


def kernel(x, edge_index, W, b):
    src = edge_index[0].astype(jnp.int32)
    dst = edge_index[1].astype(jnp.int32)
    pad = jnp.full((E_PAD - src.shape[0],), SENT, jnp.int32)
    src2 = jnp.concatenate([src, pad]).reshape(ROWS, CH)
    dst2 = jnp.concatenate([dst, pad]).reshape(ROWS, CH)
    col = jnp.arange(HW)
    ones_l = jnp.where(col < 8, 1.0, 0.0).astype(jnp.float32)
    ones_l = jnp.broadcast_to(ones_l, (CH, HW))
    ones_r = jnp.where(col >= 8, 1.0, 0.0).astype(jnp.float32)
    ones_r = jnp.broadcast_to(ones_r, (CH, HW))
    zcol = jnp.zeros((HSW, HW), jnp.float32)
    zrow = jnp.zeros((CH, DIM), jnp.float32)

    deg = _deg_kernel(src2, dst2, ones_l, ones_r, zcol)
    h, nd = _norm_call(x, deg)
    agg = _agg_kernel(h, src2, dst2, zrow)
    out = _proj_call(agg, nd, W, b.reshape(1, DIM))
    return out


# speedup vs baseline: 1.5833x; 1.5833x over previous
import functools
import jax
import jax.numpy as jnp
from jax import lax
from jax.experimental import pallas as pl
from jax.experimental.pallas import tpu as pltpu
from jax.experimental.pallas import tpu_sc as plsc

N = 100000
DIM = 32
HALF = 50000
CH = 128
ROWS = 12544
E_PAD = ROWS * CH
SENT = 100000
HPAD = 100352
HSL = HPAD // 16
HW = 16
HSW = 392
AK = 56
AG = 7
APAD = 50048
TRASH = HALF
NWCH = APAD // CH

A_ROWS = ROWS // 32
C_ROWS = ROWS // 16
CK = 28
CG = C_ROWS // CK
NB = 4
ST = CK // NB

RB = 6272

_mesh = plsc.VectorSubcoreMesh(core_axis_name="c", subcore_axis_name="s")


@functools.partial(
    pl.kernel,
    out_type=jax.ShapeDtypeStruct((2, HPAD, HW), jnp.float32),
    mesh=_mesh,
    compiler_params=pltpu.CompilerParams(use_tc_tiling_on_sc=False),
    scratch_types=[
        pltpu.VMEM((AK, CH), jnp.int32),
        pltpu.VMEM((AK, CH), jnp.int32),
        pltpu.VMEM((CH, HW), jnp.float32),
        pltpu.VMEM((CH, HW), jnp.float32),
        pltpu.VMEM((HSW, HW), jnp.float32),
        pltpu.VMEM_SHARED((HPAD, HW), jnp.float32),
    ],
)
def _deg_kernel(src_hbm, dst_hbm, onl_hbm, onr_hbm, zcol_hbm, deg_hbm,
                srcb, dstb, onlb, onrb, slabb, hist):
    c = lax.axis_index("c")
    s = lax.axis_index("s")
    w = s * 2 + c

    pltpu.sync_copy(zcol_hbm, slabb)
    for k in range(16):
        pltpu.sync_copy(slabb, hist.at[pl.ds(s * HSL + k * HSW, HSW)])
    pltpu.sync_copy(onl_hbm, onlb)
    pltpu.sync_copy(onr_hbm, onrb)
    plsc.subcore_barrier()

    def group(g, carry):
        r0 = w * A_ROWS + g * AK
        pltpu.sync_copy(src_hbm.at[pl.ds(r0, AK)], srcb)
        pltpu.sync_copy(dst_hbm.at[pl.ds(r0, AK)], dstb)

        def body(j, cc):
            pltpu.sync_copy(onlb, hist.at[srcb.at[j]], add=True)
            pltpu.sync_copy(onrb, hist.at[dstb.at[j]], add=True)
            return cc

        lax.fori_loop(0, AK, body, 0)
        return carry

    lax.fori_loop(0, AG, group, 0)
    plsc.subcore_barrier()

    for k in range(16):
        r = s * HSL + k * HSW
        pltpu.sync_copy(hist.at[pl.ds(r, HSW)], slabb)
        pltpu.sync_copy(slabb, deg_hbm.at[c, pl.ds(r, HSW)])


@functools.partial(
    pl.kernel,
    out_type=jax.ShapeDtypeStruct((N, DIM), jnp.float32),
    mesh=_mesh,
    compiler_params=pltpu.CompilerParams(use_tc_tiling_on_sc=False),
    scratch_types=(
        [pltpu.VMEM((CK, CH), jnp.int32)] * 2
        + [pltpu.VMEM((CH, DIM), jnp.float32)] * NB
        + [pltpu.SemaphoreType.DMA] * (2 * NB)
        + [pltpu.VMEM_SHARED((APAD, DIM), jnp.float32)]
    ),
)
def _agg_kernel(h_hbm, src_hbm, dst_hbm, zrow_hbm, out_hbm,
                srcb, dstb, r0b, r1b, r2b, r3b,
                g0, g1, g2, g3, s0, s1, s2, s3, acc):
    c = lax.axis_index("c")
    s = lax.axis_index("s")
    base = c * HALF
    rows = [r0b, r1b, r2b, r3b]
    gsem = [g0, g1, g2, g3]
    ssem = [s0, s1, s2, s3]

    pltpu.sync_copy(zrow_hbm, r0b)
    for k in range(25):
        ch = s + 16 * k

        @pl.when(ch < NWCH)
        def _():
            pltpu.sync_copy(r0b, acc.at[pl.ds(ch * CH, CH)])

    plsc.subcore_barrier()

    def group(g, carry):
        @pl.when(g > 0)
        def _():
            for b in range(NB):
                pltpu.make_async_copy(
                    rows[b], acc.at[dstb.at[0]], ssem[b]).wait()

        r0 = s * C_ROWS + g * CK
        pltpu.sync_copy(src_hbm.at[pl.ds(r0, CK)], srcb)
        pltpu.sync_copy(dst_hbm.at[pl.ds(r0, CK)], dstb)

        def crow(j, cc):
            def cvec(i, ci):
                d = dstb[j, pl.ds(i * 16, 16)]
                loc = d - base
                ok = (loc >= 0) & (loc < HALF)
                trash = TRASH + lax.iota(jnp.int32, 16) + 16 * (i % 2)
                dstb[j, pl.ds(i * 16, 16)] = jnp.where(ok, loc, trash)
                return ci
            return lax.fori_loop(0, CH // 16, cvec, cc)

        lax.fori_loop(0, CK, crow, 0)

        def step(st, cc):
            descs = []
            for b in range(NB):
                @pl.when(st > 0)
                def _():
                    pltpu.make_async_copy(
                        rows[b], acc.at[dstb.at[0]], ssem[b]).wait()
                descs.append(pltpu.async_copy(
                    h_hbm.at[srcb.at[st * NB + b]], rows[b], gsem[b]))
            for b in range(NB):
                descs[b].wait()
                pltpu.async_copy(rows[b], acc.at[dstb.at[st * NB + b]],
                                 ssem[b], add=True)
            return cc

        lax.fori_loop(0, ST, step, 0)
        return carry

    lax.fori_loop(0, CG, group, 0)

    for b in range(NB):
        pltpu.make_async_copy(rows[b], acc.at[dstb.at[0]], ssem[b]).wait()
    plsc.subcore_barrier()

    for k in range(25):
        ch = s + 16 * k

        @pl.when(ch < NWCH - 1)
        def _():
            pltpu.sync_copy(acc.at[pl.ds(ch * CH, CH)], r0b)
            pltpu.sync_copy(r0b, out_hbm.at[pl.ds(base + ch * CH, CH)])

        @pl.when(ch == NWCH - 1)
        def _():
            pltpu.sync_copy(acc.at[pl.ds(ch * CH, 80)],
                            r0b.at[pl.ds(0, 80)])
            pltpu.sync_copy(r0b.at[pl.ds(0, 80)],
                            out_hbm.at[pl.ds(base + ch * CH, 80)])


def _norm_body(x_ref, deg_ref, h_ref, nd_ref):
    od = deg_ref[0, :, 0:1] + deg_ref[1, :, 0:1]
    idg = deg_ref[0, :, 8:9] + deg_ref[1, :, 8:9]
    ns = jnp.where(od > 0, lax.rsqrt(jnp.maximum(od, 1.0)), 0.0)
    nd = jnp.where(idg > 0, lax.rsqrt(jnp.maximum(idg, 1.0)), 0.0)
    h_ref[...] = x_ref[...] * ns
    nd_ref[...] = nd


_norm_call = pl.pallas_call(
    _norm_body,
    grid=(16,),
    in_specs=[
        pl.BlockSpec((RB, DIM), lambda i: (i, 0)),
        pl.BlockSpec((2, RB, HW), lambda i: (0, i, 0)),
    ],
    out_specs=(
        pl.BlockSpec((RB, DIM), lambda i: (i, 0)),
        pl.BlockSpec((RB, 1), lambda i: (i, 0)),
    ),
    out_shape=(
        jax.ShapeDtypeStruct((HPAD, DIM), jnp.float32),
        jax.ShapeDtypeStruct((HPAD, 1), jnp.float32),
    ),
)


def _proj_body(agg_ref, nd_ref, w_ref, b_ref, o_ref):
    a = agg_ref[...] * nd_ref[...]
    o_ref[...] = jnp.dot(a, w_ref[...],
                         preferred_element_type=jnp.float32) + b_ref[...]


_proj_call = pl.pallas_call(
    _proj_body,
    grid=(16,),
    in_specs=[
        pl.BlockSpec((RB, DIM), lambda i: (i, 0)),
        pl.BlockSpec((RB, 1), lambda i: (i, 0)),
        pl.BlockSpec((DIM, DIM), lambda i: (0, 0)),
        pl.BlockSpec((1, DIM), lambda i: (0, 0)),
    ],
    out_specs=pl.BlockSpec((RB, DIM), lambda i: (i, 0)),
    out_shape=jax.ShapeDtypeStruct((N, DIM), jnp.float32),
)


def kernel(x, edge_index, W, b):
    src = edge_index[0].astype(jnp.int32)
    dst = edge_index[1].astype(jnp.int32)
    pad = jnp.full((E_PAD - src.shape[0],), SENT, jnp.int32)
    src2 = jnp.concatenate([src, pad]).reshape(ROWS, CH)
    dst2 = jnp.concatenate([dst, pad]).reshape(ROWS, CH)
    col = jnp.arange(HW)
    ones_l = jnp.where(col < 8, 1.0, 0.0).astype(jnp.float32)
    ones_l = jnp.broadcast_to(ones_l, (CH, HW))
    ones_r = jnp.where(col >= 8, 1.0, 0.0).astype(jnp.float32)
    ones_r = jnp.broadcast_to(ones_r, (CH, HW))
    zcol = jnp.zeros((HSW, HW), jnp.float32)
    zrow = jnp.zeros((CH, DIM), jnp.float32)

    deg = _deg_kernel(src2, dst2, ones_l, ones_r, zcol)
    h, nd = _norm_call(x, deg)
    agg = _agg_kernel(h, src2, dst2, zrow)
    out = _proj_call(agg, nd, W, b.reshape(1, DIM))
    return out


# speedup vs baseline: 1.7964x; 1.1346x over previous
import functools
import jax
import jax.numpy as jnp
from jax import lax
from jax.experimental import pallas as pl
from jax.experimental.pallas import tpu as pltpu
from jax.experimental.pallas import tpu_sc as plsc

N = 100000
DIM = 32
HALF = 50000
CH = 128
ROWS = 12500
HPAD = 100352
HSL = HPAD // 16
HW = 16
HSW = 392
AK = 13
AP = 15
APAD = 50048
TRASH = HALF
NWCH = APAD // CH

A_ROWS = 390
A_TAIL = 20
C_ROWS = 780
C_TAIL = 20
CK = 30
CG = C_ROWS // CK
CCAP = 4096
CDUMP = 4080

RB = 6272

_mesh = plsc.VectorSubcoreMesh(core_axis_name="c", subcore_axis_name="s")


@functools.partial(
    pl.kernel,
    out_type=jax.ShapeDtypeStruct((2, HPAD, HW), jnp.float32),
    mesh=_mesh,
    compiler_params=pltpu.CompilerParams(use_tc_tiling_on_sc=False,
                                         needs_layout_passes=False),
    scratch_types=[
        pltpu.VMEM((AK, CH), jnp.int32),
        pltpu.VMEM((AK, CH), jnp.int32),
        pltpu.VMEM((AK, CH), jnp.int32),
        pltpu.VMEM((AK, CH), jnp.int32),
        pltpu.VMEM((CH, HW), jnp.float32),
        pltpu.VMEM((CH, HW), jnp.float32),
        pltpu.VMEM((HSW, HW), jnp.float32),
        pltpu.SemaphoreType.DMA,
        pltpu.SemaphoreType.DMA,
        pltpu.VMEM_SHARED((HPAD, HW), jnp.float32),
    ],
)
def _deg_kernel(src_hbm, dst_hbm, onl_hbm, onr_hbm, zcol_hbm, deg_hbm,
                srcb0, dstb0, srcb1, dstb1, onlb, onrb, slabb,
                d0, d1, hist):
    c = lax.axis_index("c")
    s = lax.axis_index("s")
    w = s * 2 + c
    srcb = [srcb0, srcb1]
    dstb = [dstb0, dstb1]
    dsem = [d0, d1]

    pltpu.sync_copy(zcol_hbm, slabb)
    for k in range(16):
        pltpu.sync_copy(slabb, hist.at[pl.ds(s * HSL + k * HSW, HSW)])
    pltpu.sync_copy(onl_hbm, onlb)
    pltpu.sync_copy(onr_hbm, onrb)
    plsc.subcore_barrier()

    def drain(p, n):
        def dbody(i, cc):
            pltpu.make_async_copy(onl_hbm, onlb, dsem[p]).wait()
            return cc
        lax.fori_loop(0, n, dbody, 0)

    def pair(t, carry):
        for p in range(2):
            g = 2 * t + p

            @pl.when(t > 0)
            def _():
                drain(p, 2 * AK)

            r0 = w * A_ROWS + g * AK
            pltpu.sync_copy(src_hbm.at[pl.ds(r0, AK)], srcb[p])
            pltpu.sync_copy(dst_hbm.at[pl.ds(r0, AK)], dstb[p])

            def body(j, cc):
                pltpu.async_copy(onlb, hist.at[srcb[p].at[j]],
                                 dsem[p], add=True)
                pltpu.async_copy(onrb, hist.at[dstb[p].at[j]],
                                 dsem[p], add=True)
                return cc

            lax.fori_loop(0, AK, body, 0)
        return carry

    lax.fori_loop(0, AP, pair, 0)
    for p in range(2):
        drain(p, 2 * AK)

    @pl.when(w < A_TAIL)
    def _():
        pltpu.sync_copy(src_hbm.at[pl.ds(32 * A_ROWS + w, 1)],
                        srcb0.at[pl.ds(0, 1)])
        pltpu.sync_copy(dst_hbm.at[pl.ds(32 * A_ROWS + w, 1)],
                        dstb0.at[pl.ds(0, 1)])
        pltpu.sync_copy(onlb, hist.at[srcb0.at[0]], add=True)
        pltpu.sync_copy(onrb, hist.at[dstb0.at[0]], add=True)

    plsc.subcore_barrier()

    for k in range(16):
        r = s * HSL + k * HSW
        pltpu.sync_copy(hist.at[pl.ds(r, HSW)], slabb)
        pltpu.sync_copy(slabb, deg_hbm.at[c, pl.ds(r, HSW)])


@functools.partial(
    pl.kernel,
    out_type=jax.ShapeDtypeStruct((N, DIM), jnp.float32),
    mesh=_mesh,
    compiler_params=pltpu.CompilerParams(use_tc_tiling_on_sc=False,
                                         needs_layout_passes=False),
    scratch_types=(
        [pltpu.VMEM((CK, CH), jnp.int32)] * 2
        + [pltpu.VMEM((CCAP,), jnp.int32)] * 2
        + [pltpu.VMEM((2, CH), jnp.int32)]
        + [pltpu.VMEM((CH, DIM), jnp.float32)] * 2
        + [pltpu.SemaphoreType.DMA] * 4
        + [pltpu.VMEM_SHARED((APAD, DIM), jnp.float32)]
    ),
)
def _agg_kernel(h_hbm, src_hbm, dst_hbm, zrow_hbm, out_hbm,
                srcb, dstb, csrc, cdst, idx2d, r0b, r1b,
                g0, g1, s0s, s1s, acc):
    c = lax.axis_index("c")
    s = lax.axis_index("s")
    base = c * HALF
    rows = [r0b, r1b]
    gsem = [g0, g1]
    ssem = [s0s, s1s]

    pltpu.sync_copy(zrow_hbm, r0b)
    for k in range(25):
        ch = s + 16 * k

        @pl.when(ch < NWCH)
        def _():
            pltpu.sync_copy(r0b, acc.at[pl.ds(ch * CH, CH)])

    plsc.subcore_barrier()

    def fire(cond, carry):
        n, fired, outst, tot, so0, so1 = carry
        sos = [so0, so1]
        for b in range(2):
            @pl.when(cond & (tot % 2 == b))
            def _():
                @pl.when(outst == 1)
                def _():
                    pltpu.make_async_copy(
                        h_hbm.at[idx2d.at[1 - b]], rows[1 - b],
                        gsem[1 - b]).wait()
                    pltpu.async_copy(rows[1 - b], acc.at[idx2d.at[1 - b]],
                                     ssem[1 - b], add=True)

                @pl.when(sos[b] == 1)
                def _():
                    pltpu.make_async_copy(
                        rows[b], acc.at[idx2d.at[b]], ssem[b]).wait()

                def mvi(i, cc):
                    idx2d[b, pl.ds(i * 16, 16)] = (
                        cdst[pl.ds(fired * 128 + i * 16, 16)])
                    return cc
                lax.fori_loop(0, 8, mvi, 0)
                pltpu.async_copy(h_hbm.at[csrc.at[pl.ds(fired * 128, 128)]],
                                 rows[b], gsem[b])

        p = tot % 2
        so0 = jnp.where(cond & (p == 0), 0, so0)
        so0 = jnp.where(cond & (p == 1) & (outst == 1), 1, so0)
        so1 = jnp.where(cond & (p == 1), 0, so1)
        so1 = jnp.where(cond & (p == 0) & (outst == 1), 1, so1)
        fired = jnp.where(cond, fired + 1, fired)
        tot = jnp.where(cond, tot + 1, tot)
        outst = jnp.where(cond, 1, outst)
        return (n, fired, outst, tot, so0, so1)

    def flush(carry):
        n, fired, outst, tot, so0, so1 = carry
        for q in range(2):
            @pl.when((outst == 1) & ((tot - 1) % 2 == q))
            def _():
                pltpu.make_async_copy(
                    h_hbm.at[idx2d.at[q]], rows[q], gsem[q]).wait()
                pltpu.async_copy(rows[q], acc.at[idx2d.at[q]],
                                 ssem[q], add=True)

        q = (tot - 1) % 2
        so0 = jnp.where((outst == 1) & (q == 0), 1, so0)
        so1 = jnp.where((outst == 1) & (q == 1), 1, so1)
        return (n, fired, jnp.int32(0) * outst, tot, so0, so1)

    def row(j, carry, enable):
        n, fired, outst, tot, so0, so1 = carry

        def cvec(i, nn):
            d = dstb[j, pl.ds(i * 16, 16)]
            sv = srcb[j, pl.ds(i * 16, 16)]
            loc = d - base
            ok = (loc >= 0) & (loc < HALF) & enable
            oki = jnp.where(ok, 1, 0).astype(jnp.int32)
            cs = plsc.cumsum(oki)
            pos = jnp.where(ok, nn + cs - oki,
                            CDUMP + lax.iota(jnp.int32, 16))
            plsc.store_scatter(cdst, [pos], loc)
            plsc.store_scatter(csrc, [pos], sv)
            return nn + plsc.all_reduce_population_count(ok)

        n = lax.fori_loop(0, CH // 16, cvec, n)
        carry = (n, fired, outst, tot, so0, so1)
        return fire(jnp.any(n >= (fired + 1) * 128), carry)

    def group(g, carry):
        r0 = s * C_ROWS + g * CK
        pltpu.sync_copy(src_hbm.at[pl.ds(r0, CK)], srcb)
        pltpu.sync_copy(dst_hbm.at[pl.ds(r0, CK)], dstb)

        def rbody(j, cc):
            return row(j, cc, jnp.bool_(True))

        carry = lax.fori_loop(0, CK, rbody, carry)
        carry = flush(carry)
        n, fired, outst, tot, so0, so1 = carry

        @pl.when(fired > 0)
        def _():
            def mv(i, cc):
                csrc[pl.ds(i * 16, 16)] = csrc[pl.ds(fired * 128 + i * 16, 16)]
                cdst[pl.ds(i * 16, 16)] = cdst[pl.ds(fired * 128 + i * 16, 16)]
                return cc
            lax.fori_loop(0, 8, mv, 0)

        n = n - fired * 128
        return (n, jnp.int32(0) * fired, outst, tot, so0, so1)

    carry = (jnp.zeros((16,), jnp.int32), jnp.int32(0), jnp.int32(0),
             jnp.int32(0), jnp.int32(0), jnp.int32(0))
    carry = lax.fori_loop(0, CG, group, carry)

    pltpu.sync_copy(src_hbm.at[pl.ds(16 * C_ROWS + s, 1)],
                    srcb.at[pl.ds(0, 1)])
    pltpu.sync_copy(dst_hbm.at[pl.ds(16 * C_ROWS + s, 1)],
                    dstb.at[pl.ds(0, 1)])
    carry = row(0, carry, jnp.bool_(True))

    @pl.when(s < 4)
    def _():
        pltpu.sync_copy(src_hbm.at[pl.ds(16 * C_ROWS + 16 + s, 1)],
                        srcb.at[pl.ds(0, 1)])
        pltpu.sync_copy(dst_hbm.at[pl.ds(16 * C_ROWS + 16 + s, 1)],
                        dstb.at[pl.ds(0, 1)])

    carry = row(0, carry, s < 4)
    carry = flush(carry)
    n, fired, outst, tot, so0, so1 = carry

    for k in range(8):
        idxv = n + 16 * k + lax.iota(jnp.int32, 16)
        plsc.store_scatter(cdst, [idxv],
                           TRASH + lax.iota(jnp.int32, 16) + 16 * (k % 3))
        plsc.store_scatter(csrc, [idxv], jnp.zeros((16,), jnp.int32))

    rem = n - fired * 128
    carry = fire(jnp.any(rem > 0), (n, fired, outst, tot, so0, so1))
    carry = flush(carry)
    n, fired, outst, tot, so0, so1 = carry
    sos = [so0, so1]
    for q in range(2):
        @pl.when(sos[q] == 1)
        def _():
            pltpu.make_async_copy(rows[q], acc.at[idx2d.at[q]],
                                  ssem[q]).wait()

    plsc.subcore_barrier()

    for k in range(25):
        ch = s + 16 * k

        @pl.when(ch < NWCH - 1)
        def _():
            pltpu.sync_copy(acc.at[pl.ds(ch * CH, CH)], r0b)
            pltpu.sync_copy(r0b, out_hbm.at[pl.ds(base + ch * CH, CH)])

        @pl.when(ch == NWCH - 1)
        def _():
            pltpu.sync_copy(acc.at[pl.ds(ch * CH, 80)],
                            r0b.at[pl.ds(0, 80)])
            pltpu.sync_copy(r0b.at[pl.ds(0, 80)],
                            out_hbm.at[pl.ds(base + ch * CH, 80)])


def _norm_body(x_ref, deg_ref, h_ref, nd_ref):
    od = deg_ref[0, :, 0:1] + deg_ref[1, :, 0:1]
    idg = deg_ref[0, :, 8:9] + deg_ref[1, :, 8:9]
    ns = jnp.where(od > 0, lax.rsqrt(jnp.maximum(od, 1.0)), 0.0)
    nd = jnp.where(idg > 0, lax.rsqrt(jnp.maximum(idg, 1.0)), 0.0)
    h_ref[...] = x_ref[...] * ns
    nd_ref[...] = nd


_norm_call = pl.pallas_call(
    _norm_body,
    grid=(16,),
    in_specs=[
        pl.BlockSpec((RB, DIM), lambda i: (i, 0)),
        pl.BlockSpec((2, RB, HW), lambda i: (0, i, 0)),
    ],
    out_specs=(
        pl.BlockSpec((RB, DIM), lambda i: (i, 0)),
        pl.BlockSpec((RB, 1), lambda i: (i, 0)),
    ),
    out_shape=(
        jax.ShapeDtypeStruct((HPAD, DIM), jnp.float32),
        jax.ShapeDtypeStruct((HPAD, 1), jnp.float32),
    ),
)


def _proj_body(agg_ref, nd_ref, w_ref, b_ref, o_ref):
    a = agg_ref[...] * nd_ref[...]
    o_ref[...] = jnp.dot(a, w_ref[...],
                         preferred_element_type=jnp.float32) + b_ref[...]


_proj_call = pl.pallas_call(
    _proj_body,
    grid=(16,),
    in_specs=[
        pl.BlockSpec((RB, DIM), lambda i: (i, 0)),
        pl.BlockSpec((RB, 1), lambda i: (i, 0)),
        pl.BlockSpec((DIM, DIM), lambda i: (0, 0)),
        pl.BlockSpec((1, DIM), lambda i: (0, 0)),
    ],
    out_specs=pl.BlockSpec((RB, DIM), lambda i: (i, 0)),
    out_shape=jax.ShapeDtypeStruct((N, DIM), jnp.float32),
)


def kernel(x, edge_index, W, b):
    src2 = edge_index[0].astype(jnp.int32).reshape(ROWS, CH)
    dst2 = edge_index[1].astype(jnp.int32).reshape(ROWS, CH)
    col = jnp.arange(HW)
    ones_l = jnp.where(col < 8, 1.0, 0.0).astype(jnp.float32)
    ones_l = jnp.broadcast_to(ones_l, (CH, HW))
    ones_r = jnp.where(col >= 8, 1.0, 0.0).astype(jnp.float32)
    ones_r = jnp.broadcast_to(ones_r, (CH, HW))
    zcol = jnp.zeros((HSW, HW), jnp.float32)
    zrow = jnp.zeros((CH, DIM), jnp.float32)

    deg = _deg_kernel(src2, dst2, ones_l, ones_r, zcol)
    h, nd = _norm_call(x, deg)
    agg = _agg_kernel(h, src2, dst2, zrow)
    out = _proj_call(agg, nd, W, b.reshape(1, DIM))
    return out


# speedup vs baseline: 2.1339x; 1.1879x over previous
import functools
import jax
import jax.numpy as jnp
from jax import lax
from jax.experimental import pallas as pl
from jax.experimental.pallas import tpu as pltpu
from jax.experimental.pallas import tpu_sc as plsc

N = 100000
DIM = 32
HALF = 50000
CH = 128
ROWS = 12500
HPAD = 100352
HSL = HPAD // 16
HW = 16
HSW = 392
AK = 13
AP = 15
APAD = 50048
TRASH = HALF
NWCH = APAD // CH

A_ROWS = 390
A_TAIL = 20
C_ROWS = 780
C_TAIL = 20
CK = 30
CG = C_ROWS // CK
CCAP = 4096
CDUMP = 4080

RB = 6272

_mesh = plsc.VectorSubcoreMesh(core_axis_name="c", subcore_axis_name="s")


@functools.partial(
    pl.kernel,
    out_type=jax.ShapeDtypeStruct((2, HPAD, HW), jnp.float32),
    mesh=_mesh,
    compiler_params=pltpu.CompilerParams(use_tc_tiling_on_sc=False,
                                         needs_layout_passes=False),
    scratch_types=[
        pltpu.VMEM((AK, CH), jnp.int32),
        pltpu.VMEM((AK, CH), jnp.int32),
        pltpu.VMEM((AK, CH), jnp.int32),
        pltpu.VMEM((AK, CH), jnp.int32),
        pltpu.VMEM((CH, HW), jnp.float32),
        pltpu.VMEM((CH, HW), jnp.float32),
        pltpu.VMEM((HSW, HW), jnp.float32),
        pltpu.SemaphoreType.DMA,
        pltpu.SemaphoreType.DMA,
        pltpu.VMEM_SHARED((HPAD, HW), jnp.float32),
    ],
)
def _deg_kernel(src_hbm, dst_hbm, onl_hbm, onr_hbm, zcol_hbm, deg_hbm,
                srcb0, dstb0, srcb1, dstb1, onlb, onrb, slabb,
                d0, d1, hist):
    c = lax.axis_index("c")
    s = lax.axis_index("s")
    w = s * 2 + c
    srcb = [srcb0, srcb1]
    dstb = [dstb0, dstb1]
    dsem = [d0, d1]

    pltpu.sync_copy(zcol_hbm, slabb)
    for k in range(16):
        pltpu.sync_copy(slabb, hist.at[pl.ds(s * HSL + k * HSW, HSW)])
    pltpu.sync_copy(onl_hbm, onlb)
    pltpu.sync_copy(onr_hbm, onrb)
    plsc.subcore_barrier()

    def drain(p, n):
        def dbody(i, cc):
            pltpu.make_async_copy(onl_hbm, onlb, dsem[p]).wait()
            return cc
        lax.fori_loop(0, n, dbody, 0)

    def pair(t, carry):
        for p in range(2):
            g = 2 * t + p

            @pl.when(t > 0)
            def _():
                drain(p, 2 * AK)

            r0 = w * A_ROWS + g * AK
            pltpu.sync_copy(src_hbm.at[pl.ds(r0, AK)], srcb[p])
            pltpu.sync_copy(dst_hbm.at[pl.ds(r0, AK)], dstb[p])

            def body(j, cc):
                pltpu.async_copy(onlb, hist.at[srcb[p].at[j]],
                                 dsem[p], add=True)
                pltpu.async_copy(onrb, hist.at[dstb[p].at[j]],
                                 dsem[p], add=True)
                return cc

            lax.fori_loop(0, AK, body, 0)
        return carry

    lax.fori_loop(0, AP, pair, 0)
    for p in range(2):
        drain(p, 2 * AK)

    @pl.when(w < A_TAIL)
    def _():
        pltpu.sync_copy(src_hbm.at[pl.ds(32 * A_ROWS + w, 1)],
                        srcb0.at[pl.ds(0, 1)])
        pltpu.sync_copy(dst_hbm.at[pl.ds(32 * A_ROWS + w, 1)],
                        dstb0.at[pl.ds(0, 1)])
        pltpu.sync_copy(onlb, hist.at[srcb0.at[0]], add=True)
        pltpu.sync_copy(onrb, hist.at[dstb0.at[0]], add=True)

    plsc.subcore_barrier()

    for k in range(16):
        r = s * HSL + k * HSW
        pltpu.sync_copy(hist.at[pl.ds(r, HSW)], slabb)
        pltpu.sync_copy(slabb, deg_hbm.at[c, pl.ds(r, HSW)])


@functools.partial(
    pl.kernel,
    out_type=jax.ShapeDtypeStruct((N, DIM), jnp.float32),
    mesh=_mesh,
    compiler_params=pltpu.CompilerParams(use_tc_tiling_on_sc=False,
                                         needs_layout_passes=False),
    scratch_types=(
        [pltpu.VMEM((CK, CH), jnp.int32)] * 2
        + [pltpu.VMEM((CCAP,), jnp.int32)] * 2
        + [pltpu.VMEM((3, CH), jnp.int32)]
        + [pltpu.VMEM((CH, DIM), jnp.float32)] * 3
        + [pltpu.SemaphoreType.DMA] * 6
        + [pltpu.VMEM_SHARED((APAD, DIM), jnp.float32)]
    ),
)
def _agg_kernel(h_hbm, src_hbm, dst_hbm, zrow_hbm, out_hbm,
                srcb, dstb, csrc, cdst, idx2d, r0b, r1b, r2b,
                g0, g1, g2, s0s, s1s, s2s, acc):
    c = lax.axis_index("c")
    s = lax.axis_index("s")
    base = c * HALF
    rows = [r0b, r1b, r2b]
    gsem = [g0, g1, g2]
    ssem = [s0s, s1s, s2s]

    pltpu.sync_copy(zrow_hbm, r0b)
    for k in range(25):
        ch = s + 16 * k

        @pl.when(ch < NWCH)
        def _():
            pltpu.sync_copy(r0b, acc.at[pl.ds(ch * CH, CH)])

    plsc.subcore_barrier()

    def retire(q):
        pltpu.make_async_copy(h_hbm.at[idx2d.at[q]], rows[q],
                              gsem[q]).wait()
        pltpu.async_copy(rows[q], acc.at[idx2d.at[q]], ssem[q], add=True)

    def fire(cond, carry):
        n, fired, ost, tot, so0, so1, so2 = carry
        sos = [so0, so1, so2]
        for b in range(3):
            @pl.when(cond & (tot % 3 == b))
            def _():
                q = (b + 1) % 3

                @pl.when(ost >= 2)
                def _():
                    retire(q)

                @pl.when(sos[b] == 1)
                def _():
                    pltpu.make_async_copy(
                        rows[b], acc.at[idx2d.at[b]], ssem[b]).wait()

                def mvi(i, cc):
                    idx2d[b, pl.ds(i * 16, 16)] = (
                        cdst[pl.ds(fired * 128 + i * 16, 16)])
                    return cc
                lax.fori_loop(0, 8, mvi, 0)
                pltpu.async_copy(h_hbm.at[csrc.at[pl.ds(fired * 128, 128)]],
                                 rows[b], gsem[b])

        p = tot % 3
        qd = (tot - 2) % 3
        rearm = cond & (ost >= 2)
        so0 = jnp.where(rearm & (qd == 0), 1, so0)
        so1 = jnp.where(rearm & (qd == 1), 1, so1)
        so2 = jnp.where(rearm & (qd == 2), 1, so2)
        so0 = jnp.where(cond & (p == 0), 0, so0)
        so1 = jnp.where(cond & (p == 1), 0, so1)
        so2 = jnp.where(cond & (p == 2), 0, so2)
        fired = jnp.where(cond, fired + 1, fired)
        tot = jnp.where(cond, tot + 1, tot)
        ost = jnp.where(cond, jnp.minimum(ost + 1, 2), ost)
        return (n, fired, ost, tot, so0, so1, so2)

    def flush(carry):
        n, fired, ost, tot, so0, so1, so2 = carry
        for age in (2, 1):
            for q in range(3):
                @pl.when((ost >= age) & ((tot - age) % 3 == q))
                def _():
                    retire(q)

            qa = (tot - age) % 3
            so0 = jnp.where((ost >= age) & (qa == 0), 1, so0)
            so1 = jnp.where((ost >= age) & (qa == 1), 1, so1)
            so2 = jnp.where((ost >= age) & (qa == 2), 1, so2)
        return (n, fired, jnp.int32(0) * ost, tot, so0, so1, so2)

    def row(j, carry, enable):
        n = carry[0]

        def cvec(i, nn):
            d = dstb[j, pl.ds(i * 16, 16)]
            sv = srcb[j, pl.ds(i * 16, 16)]
            loc = d - base
            ok = (loc >= 0) & (loc < HALF) & enable
            oki = jnp.where(ok, 1, 0).astype(jnp.int32)
            cs = plsc.cumsum(oki)
            pos = jnp.where(ok, nn + cs - oki,
                            CDUMP + lax.iota(jnp.int32, 16))
            plsc.store_scatter(cdst, [pos], loc)
            plsc.store_scatter(csrc, [pos], sv)
            return nn + plsc.all_reduce_population_count(ok)

        n = lax.fori_loop(0, CH // 16, cvec, n)
        carry = (n,) + carry[1:]
        return fire(jnp.any(n >= (carry[1] + 1) * 128), carry)

    def group(g, carry):
        r0 = s * C_ROWS + g * CK
        pltpu.sync_copy(src_hbm.at[pl.ds(r0, CK)], srcb)
        pltpu.sync_copy(dst_hbm.at[pl.ds(r0, CK)], dstb)

        def rbody(j, cc):
            return row(j, cc, jnp.bool_(True))

        carry = lax.fori_loop(0, CK, rbody, carry)
        carry = flush(carry)
        n, fired = carry[0], carry[1]

        @pl.when(fired > 0)
        def _():
            def mv(i, cc):
                csrc[pl.ds(i * 16, 16)] = csrc[pl.ds(fired * 128 + i * 16, 16)]
                cdst[pl.ds(i * 16, 16)] = cdst[pl.ds(fired * 128 + i * 16, 16)]
                return cc
            lax.fori_loop(0, 8, mv, 0)

        return (n - fired * 128, jnp.int32(0) * fired) + carry[2:]

    carry = (jnp.zeros((16,), jnp.int32), jnp.int32(0), jnp.int32(0),
             jnp.int32(0), jnp.int32(0), jnp.int32(0), jnp.int32(0))
    carry = lax.fori_loop(0, CG, group, carry)

    pltpu.sync_copy(src_hbm.at[pl.ds(16 * C_ROWS + s, 1)],
                    srcb.at[pl.ds(0, 1)])
    pltpu.sync_copy(dst_hbm.at[pl.ds(16 * C_ROWS + s, 1)],
                    dstb.at[pl.ds(0, 1)])
    carry = row(0, carry, jnp.bool_(True))

    @pl.when(s < 4)
    def _():
        pltpu.sync_copy(src_hbm.at[pl.ds(16 * C_ROWS + 16 + s, 1)],
                        srcb.at[pl.ds(0, 1)])
        pltpu.sync_copy(dst_hbm.at[pl.ds(16 * C_ROWS + 16 + s, 1)],
                        dstb.at[pl.ds(0, 1)])

    carry = row(0, carry, s < 4)
    carry = flush(carry)
    n, fired = carry[0], carry[1]

    for k in range(8):
        idxv = n + 16 * k + lax.iota(jnp.int32, 16)
        plsc.store_scatter(cdst, [idxv],
                           TRASH + lax.iota(jnp.int32, 16) + 16 * (k % 3))
        plsc.store_scatter(csrc, [idxv], jnp.zeros((16,), jnp.int32))

    carry = fire(jnp.any(n - fired * 128 > 0), carry)
    carry = flush(carry)
    sos = [carry[4], carry[5], carry[6]]
    for q in range(3):
        @pl.when(sos[q] == 1)
        def _():
            pltpu.make_async_copy(rows[q], acc.at[idx2d.at[q]],
                                  ssem[q]).wait()

    plsc.subcore_barrier()

    for k in range(25):
        ch = s + 16 * k

        @pl.when(ch < NWCH - 1)
        def _():
            pltpu.sync_copy(acc.at[pl.ds(ch * CH, CH)], r0b)
            pltpu.sync_copy(r0b, out_hbm.at[pl.ds(base + ch * CH, CH)])

        @pl.when(ch == NWCH - 1)
        def _():
            pltpu.sync_copy(acc.at[pl.ds(ch * CH, 80)],
                            r0b.at[pl.ds(0, 80)])
            pltpu.sync_copy(r0b.at[pl.ds(0, 80)],
                            out_hbm.at[pl.ds(base + ch * CH, 80)])


def _norm_body(x_ref, deg_ref, h_ref, nd_ref):
    od = deg_ref[0, :, 0:1] + deg_ref[1, :, 0:1]
    idg = deg_ref[0, :, 8:9] + deg_ref[1, :, 8:9]
    ns = jnp.where(od > 0, lax.rsqrt(jnp.maximum(od, 1.0)), 0.0)
    nd = jnp.where(idg > 0, lax.rsqrt(jnp.maximum(idg, 1.0)), 0.0)
    h_ref[...] = x_ref[...] * ns
    nd_ref[...] = nd


_norm_call = pl.pallas_call(
    _norm_body,
    grid=(16,),
    in_specs=[
        pl.BlockSpec((RB, DIM), lambda i: (i, 0)),
        pl.BlockSpec((2, RB, HW), lambda i: (0, i, 0)),
    ],
    out_specs=(
        pl.BlockSpec((RB, DIM), lambda i: (i, 0)),
        pl.BlockSpec((RB, 1), lambda i: (i, 0)),
    ),
    out_shape=(
        jax.ShapeDtypeStruct((HPAD, DIM), jnp.float32),
        jax.ShapeDtypeStruct((HPAD, 1), jnp.float32),
    ),
)


def _proj_body(agg_ref, nd_ref, w_ref, b_ref, o_ref):
    a = agg_ref[...] * nd_ref[...]
    o_ref[...] = jnp.dot(a, w_ref[...],
                         preferred_element_type=jnp.float32) + b_ref[...]


_proj_call = pl.pallas_call(
    _proj_body,
    grid=(16,),
    in_specs=[
        pl.BlockSpec((RB, DIM), lambda i: (i, 0)),
        pl.BlockSpec((RB, 1), lambda i: (i, 0)),
        pl.BlockSpec((DIM, DIM), lambda i: (0, 0)),
        pl.BlockSpec((1, DIM), lambda i: (0, 0)),
    ],
    out_specs=pl.BlockSpec((RB, DIM), lambda i: (i, 0)),
    out_shape=jax.ShapeDtypeStruct((N, DIM), jnp.float32),
)


def kernel(x, edge_index, W, b):
    src2 = edge_index[0].astype(jnp.int32).reshape(ROWS, CH)
    dst2 = edge_index[1].astype(jnp.int32).reshape(ROWS, CH)
    col = jnp.arange(HW)
    ones_l = jnp.where(col < 8, 1.0, 0.0).astype(jnp.float32)
    ones_l = jnp.broadcast_to(ones_l, (CH, HW))
    ones_r = jnp.where(col >= 8, 1.0, 0.0).astype(jnp.float32)
    ones_r = jnp.broadcast_to(ones_r, (CH, HW))
    zcol = jnp.zeros((HSW, HW), jnp.float32)
    zrow = jnp.zeros((CH, DIM), jnp.float32)

    deg = _deg_kernel(src2, dst2, ones_l, ones_r, zcol)
    h, nd = _norm_call(x, deg)
    agg = _agg_kernel(h, src2, dst2, zrow)
    out = _proj_call(agg, nd, W, b.reshape(1, DIM))
    return out


# speedup vs baseline: 2.1677x; 1.0159x over previous
import functools
import jax
import jax.numpy as jnp
from jax import lax
from jax.experimental import pallas as pl
from jax.experimental.pallas import tpu as pltpu
from jax.experimental.pallas import tpu_sc as plsc

N = 100000
DIM = 32
HALF = 50000
CH = 128
ROWS = 12500
HPAD = 100352
HSL = HPAD // 16
HW = 16
HSW = 392
AK = 13
AP = 15
APAD = 50048
TRASH = HALF
NWCH = APAD // CH

A_ROWS = 390
A_TAIL = 20
C_ROWS = 780
C_TAIL = 20
CK = 26
CG = C_ROWS // CK
CCAP = 3584
CDUMP = 3568

RB = 6272

_mesh = plsc.VectorSubcoreMesh(core_axis_name="c", subcore_axis_name="s")


@functools.partial(
    pl.kernel,
    out_type=jax.ShapeDtypeStruct((2, HPAD, HW), jnp.float32),
    mesh=_mesh,
    compiler_params=pltpu.CompilerParams(use_tc_tiling_on_sc=False,
                                         needs_layout_passes=False),
    scratch_types=[
        pltpu.VMEM((AK, CH), jnp.int32),
        pltpu.VMEM((AK, CH), jnp.int32),
        pltpu.VMEM((AK, CH), jnp.int32),
        pltpu.VMEM((AK, CH), jnp.int32),
        pltpu.VMEM((CH, HW), jnp.float32),
        pltpu.VMEM((CH, HW), jnp.float32),
        pltpu.VMEM((HSW, HW), jnp.float32),
        pltpu.SemaphoreType.DMA,
        pltpu.SemaphoreType.DMA,
        pltpu.VMEM_SHARED((HPAD, HW), jnp.float32),
    ],
)
def _deg_kernel(src_hbm, dst_hbm, onl_hbm, onr_hbm, zcol_hbm, deg_hbm,
                srcb0, dstb0, srcb1, dstb1, onlb, onrb, slabb,
                d0, d1, hist):
    c = lax.axis_index("c")
    s = lax.axis_index("s")
    w = s * 2 + c
    srcb = [srcb0, srcb1]
    dstb = [dstb0, dstb1]
    dsem = [d0, d1]

    pltpu.sync_copy(zcol_hbm, slabb)
    for k in range(16):
        pltpu.sync_copy(slabb, hist.at[pl.ds(s * HSL + k * HSW, HSW)])
    pltpu.sync_copy(onl_hbm, onlb)
    pltpu.sync_copy(onr_hbm, onrb)
    plsc.subcore_barrier()

    def drain(p, n):
        def dbody(i, cc):
            pltpu.make_async_copy(onl_hbm, onlb, dsem[p]).wait()
            return cc
        lax.fori_loop(0, n, dbody, 0)

    def pair(t, carry):
        for p in range(2):
            g = 2 * t + p

            @pl.when(t > 0)
            def _():
                drain(p, 2 * AK)

            r0 = w * A_ROWS + g * AK
            pltpu.sync_copy(src_hbm.at[pl.ds(r0, AK)], srcb[p])
            pltpu.sync_copy(dst_hbm.at[pl.ds(r0, AK)], dstb[p])

            def body(j, cc):
                pltpu.async_copy(onlb, hist.at[srcb[p].at[j]],
                                 dsem[p], add=True)
                pltpu.async_copy(onrb, hist.at[dstb[p].at[j]],
                                 dsem[p], add=True)
                return cc

            lax.fori_loop(0, AK, body, 0)
        return carry

    lax.fori_loop(0, AP, pair, 0)
    for p in range(2):
        drain(p, 2 * AK)

    @pl.when(w < A_TAIL)
    def _():
        pltpu.sync_copy(src_hbm.at[pl.ds(32 * A_ROWS + w, 1)],
                        srcb0.at[pl.ds(0, 1)])
        pltpu.sync_copy(dst_hbm.at[pl.ds(32 * A_ROWS + w, 1)],
                        dstb0.at[pl.ds(0, 1)])
        pltpu.sync_copy(onlb, hist.at[srcb0.at[0]], add=True)
        pltpu.sync_copy(onrb, hist.at[dstb0.at[0]], add=True)

    plsc.subcore_barrier()

    for k in range(16):
        r = s * HSL + k * HSW
        pltpu.sync_copy(hist.at[pl.ds(r, HSW)], slabb)
        pltpu.sync_copy(slabb, deg_hbm.at[c, pl.ds(r, HSW)])


@functools.partial(
    pl.kernel,
    out_type=jax.ShapeDtypeStruct((N, DIM), jnp.float32),
    mesh=_mesh,
    compiler_params=pltpu.CompilerParams(use_tc_tiling_on_sc=False,
                                         needs_layout_passes=False),
    scratch_types=(
        [pltpu.VMEM((CK, CH), jnp.int32)] * 2
        + [pltpu.VMEM((CCAP,), jnp.int32)] * 2
        + [pltpu.VMEM((4, CH), jnp.int32)]
        + [pltpu.VMEM((CH, DIM), jnp.float32)] * 4
        + [pltpu.SemaphoreType.DMA] * 8
        + [pltpu.VMEM_SHARED((APAD, DIM), jnp.float32)]
    ),
)
def _agg_kernel(h_hbm, src_hbm, dst_hbm, zrow_hbm, out_hbm,
                srcb, dstb, csrc, cdst, idx2d, r0b, r1b, r2b, r3b,
                g0, g1, g2, g3, s0s, s1s, s2s, s3s, acc):
    c = lax.axis_index("c")
    s = lax.axis_index("s")
    base = c * HALF
    rows = [r0b, r1b, r2b, r3b]
    gsem = [g0, g1, g2, g3]
    ssem = [s0s, s1s, s2s, s3s]

    pltpu.sync_copy(zrow_hbm, r0b)
    for k in range(25):
        ch = s + 16 * k

        @pl.when(ch < NWCH)
        def _():
            pltpu.sync_copy(r0b, acc.at[pl.ds(ch * CH, CH)])

    plsc.subcore_barrier()

    def retire(q):
        pltpu.make_async_copy(h_hbm.at[idx2d.at[q]], rows[q],
                              gsem[q]).wait()
        pltpu.async_copy(rows[q], acc.at[idx2d.at[q]], ssem[q], add=True)

    def fire(cond, carry):
        n, fired, ost, tot, so0, so1, so2, so3 = carry
        sos = [so0, so1, so2, so3]
        for b in range(4):
            @pl.when(cond & (tot % 4 == b))
            def _():
                q = (b + 1) % 4

                @pl.when(ost >= 3)
                def _():
                    retire(q)

                @pl.when(sos[b] == 1)
                def _():
                    pltpu.make_async_copy(
                        rows[b], acc.at[idx2d.at[b]], ssem[b]).wait()

                def mvi(i, cc):
                    idx2d[b, pl.ds(i * 16, 16)] = (
                        cdst[pl.ds(fired * 128 + i * 16, 16)])
                    return cc
                lax.fori_loop(0, 8, mvi, 0)
                pltpu.async_copy(h_hbm.at[csrc.at[pl.ds(fired * 128, 128)]],
                                 rows[b], gsem[b])

        p = tot % 4
        qd = (tot - 3) % 4
        rearm = cond & (ost >= 3)
        so0 = jnp.where(rearm & (qd == 0), 1, so0)
        so1 = jnp.where(rearm & (qd == 1), 1, so1)
        so2 = jnp.where(rearm & (qd == 2), 1, so2)
        so3 = jnp.where(rearm & (qd == 3), 1, so3)
        so0 = jnp.where(cond & (p == 0), 0, so0)
        so1 = jnp.where(cond & (p == 1), 0, so1)
        so2 = jnp.where(cond & (p == 2), 0, so2)
        so3 = jnp.where(cond & (p == 3), 0, so3)
        fired = jnp.where(cond, fired + 1, fired)
        tot = jnp.where(cond, tot + 1, tot)
        ost = jnp.where(cond, jnp.minimum(ost + 1, 3), ost)
        return (n, fired, ost, tot, so0, so1, so2, so3)

    def flush(carry):
        n, fired, ost, tot, so0, so1, so2, so3 = carry
        for age in (3, 2, 1):
            for q in range(4):
                @pl.when((ost >= age) & ((tot - age) % 4 == q))
                def _():
                    retire(q)

            qa = (tot - age) % 4
            so0 = jnp.where((ost >= age) & (qa == 0), 1, so0)
            so1 = jnp.where((ost >= age) & (qa == 1), 1, so1)
            so2 = jnp.where((ost >= age) & (qa == 2), 1, so2)
            so3 = jnp.where((ost >= age) & (qa == 3), 1, so3)
        return (n, fired, jnp.int32(0) * ost, tot, so0, so1, so2, so3)

    def row(j, carry, enable):
        n = carry[0]

        def cvec(i, nn):
            d = dstb[j, pl.ds(i * 16, 16)]
            sv = srcb[j, pl.ds(i * 16, 16)]
            loc = d - base
            ok = (loc >= 0) & (loc < HALF) & enable
            oki = jnp.where(ok, 1, 0).astype(jnp.int32)
            cs = plsc.cumsum(oki)
            pos = jnp.where(ok, nn + cs - oki,
                            CDUMP + lax.iota(jnp.int32, 16))
            plsc.store_scatter(cdst, [pos], loc)
            plsc.store_scatter(csrc, [pos], sv)
            return nn + plsc.all_reduce_population_count(ok)

        n = lax.fori_loop(0, CH // 16, cvec, n)
        carry = (n,) + carry[1:]
        return fire(jnp.any(n >= (carry[1] + 1) * 128), carry)

    def group(g, carry):
        r0 = s * C_ROWS + g * CK
        pltpu.sync_copy(src_hbm.at[pl.ds(r0, CK)], srcb)
        pltpu.sync_copy(dst_hbm.at[pl.ds(r0, CK)], dstb)

        def rbody(j, cc):
            return row(j, cc, jnp.bool_(True))

        carry = lax.fori_loop(0, CK, rbody, carry)
        carry = flush(carry)
        n, fired = carry[0], carry[1]

        @pl.when(fired > 0)
        def _():
            def mv(i, cc):
                csrc[pl.ds(i * 16, 16)] = csrc[pl.ds(fired * 128 + i * 16, 16)]
                cdst[pl.ds(i * 16, 16)] = cdst[pl.ds(fired * 128 + i * 16, 16)]
                return cc
            lax.fori_loop(0, 8, mv, 0)

        return (n - fired * 128, jnp.int32(0) * fired) + carry[2:]

    carry = (jnp.zeros((16,), jnp.int32), jnp.int32(0), jnp.int32(0),
             jnp.int32(0), jnp.int32(0), jnp.int32(0), jnp.int32(0),
             jnp.int32(0))
    carry = lax.fori_loop(0, CG, group, carry)

    pltpu.sync_copy(src_hbm.at[pl.ds(16 * C_ROWS + s, 1)],
                    srcb.at[pl.ds(0, 1)])
    pltpu.sync_copy(dst_hbm.at[pl.ds(16 * C_ROWS + s, 1)],
                    dstb.at[pl.ds(0, 1)])
    carry = row(0, carry, jnp.bool_(True))

    @pl.when(s < 4)
    def _():
        pltpu.sync_copy(src_hbm.at[pl.ds(16 * C_ROWS + 16 + s, 1)],
                        srcb.at[pl.ds(0, 1)])
        pltpu.sync_copy(dst_hbm.at[pl.ds(16 * C_ROWS + 16 + s, 1)],
                        dstb.at[pl.ds(0, 1)])

    carry = row(0, carry, s < 4)
    carry = flush(carry)
    n, fired = carry[0], carry[1]

    for k in range(8):
        idxv = n + 16 * k + lax.iota(jnp.int32, 16)
        plsc.store_scatter(cdst, [idxv],
                           TRASH + lax.iota(jnp.int32, 16) + 16 * (k % 3))
        plsc.store_scatter(csrc, [idxv], jnp.zeros((16,), jnp.int32))

    carry = fire(jnp.any(n - fired * 128 > 0), carry)
    carry = flush(carry)
    sos = [carry[4], carry[5], carry[6], carry[7]]
    for q in range(4):
        @pl.when(sos[q] == 1)
        def _():
            pltpu.make_async_copy(rows[q], acc.at[idx2d.at[q]],
                                  ssem[q]).wait()

    plsc.subcore_barrier()

    for k in range(25):
        ch = s + 16 * k

        @pl.when(ch < NWCH - 1)
        def _():
            pltpu.sync_copy(acc.at[pl.ds(ch * CH, CH)], r0b)
            pltpu.sync_copy(r0b, out_hbm.at[pl.ds(base + ch * CH, CH)])

        @pl.when(ch == NWCH - 1)
        def _():
            pltpu.sync_copy(acc.at[pl.ds(ch * CH, 80)],
                            r0b.at[pl.ds(0, 80)])
            pltpu.sync_copy(r0b.at[pl.ds(0, 80)],
                            out_hbm.at[pl.ds(base + ch * CH, 80)])


def _norm_body(x_ref, deg_ref, h_ref, nd_ref):
    od = deg_ref[0, :, 0:1] + deg_ref[1, :, 0:1]
    idg = deg_ref[0, :, 8:9] + deg_ref[1, :, 8:9]
    ns = jnp.where(od > 0, lax.rsqrt(jnp.maximum(od, 1.0)), 0.0)
    nd = jnp.where(idg > 0, lax.rsqrt(jnp.maximum(idg, 1.0)), 0.0)
    h_ref[...] = x_ref[...] * ns
    nd_ref[...] = nd


_norm_call = pl.pallas_call(
    _norm_body,
    grid=(16,),
    in_specs=[
        pl.BlockSpec((RB, DIM), lambda i: (i, 0)),
        pl.BlockSpec((2, RB, HW), lambda i: (0, i, 0)),
    ],
    out_specs=(
        pl.BlockSpec((RB, DIM), lambda i: (i, 0)),
        pl.BlockSpec((RB, 1), lambda i: (i, 0)),
    ),
    out_shape=(
        jax.ShapeDtypeStruct((HPAD, DIM), jnp.float32),
        jax.ShapeDtypeStruct((HPAD, 1), jnp.float32),
    ),
)


def _proj_body(agg_ref, nd_ref, w_ref, b_ref, o_ref):
    a = agg_ref[...] * nd_ref[...]
    o_ref[...] = jnp.dot(a, w_ref[...],
                         preferred_element_type=jnp.float32) + b_ref[...]


_proj_call = pl.pallas_call(
    _proj_body,
    grid=(16,),
    in_specs=[
        pl.BlockSpec((RB, DIM), lambda i: (i, 0)),
        pl.BlockSpec((RB, 1), lambda i: (i, 0)),
        pl.BlockSpec((DIM, DIM), lambda i: (0, 0)),
        pl.BlockSpec((1, DIM), lambda i: (0, 0)),
    ],
    out_specs=pl.BlockSpec((RB, DIM), lambda i: (i, 0)),
    out_shape=jax.ShapeDtypeStruct((N, DIM), jnp.float32),
)


def kernel(x, edge_index, W, b):
    src2 = edge_index[0].astype(jnp.int32).reshape(ROWS, CH)
    dst2 = edge_index[1].astype(jnp.int32).reshape(ROWS, CH)
    col = jnp.arange(HW)
    ones_l = jnp.where(col < 8, 1.0, 0.0).astype(jnp.float32)
    ones_l = jnp.broadcast_to(ones_l, (CH, HW))
    ones_r = jnp.where(col >= 8, 1.0, 0.0).astype(jnp.float32)
    ones_r = jnp.broadcast_to(ones_r, (CH, HW))
    zcol = jnp.zeros((HSW, HW), jnp.float32)
    zrow = jnp.zeros((CH, DIM), jnp.float32)

    deg = _deg_kernel(src2, dst2, ones_l, ones_r, zcol)
    h, nd = _norm_call(x, deg)
    agg = _agg_kernel(h, src2, dst2, zrow)
    out = _proj_call(agg, nd, W, b.reshape(1, DIM))
    return out


# speedup vs baseline: 2.2271x; 1.0274x over previous
import functools
import jax
import jax.numpy as jnp
from jax import lax
from jax.experimental import pallas as pl
from jax.experimental.pallas import tpu as pltpu
from jax.experimental.pallas import tpu_sc as plsc

N = 100000
DIM = 32
HALF = 50000
CH = 128
ROWS = 12500
HPAD = 100352
HSL = HPAD // 16
HW = 8
HSW = 392
AK = 13
AP = 15
APAD = 50048
TRASH = HALF
NWCH = APAD // CH

A_ROWS = 390
A_TAIL = 20
C_ROWS = 780
C_TAIL = 20
CK = 26
CG = C_ROWS // CK
CCAP = 3584
CDUMP = 3568

RB = 6272

_mesh = plsc.VectorSubcoreMesh(core_axis_name="c", subcore_axis_name="s")


@functools.partial(
    pl.kernel,
    out_type=jax.ShapeDtypeStruct((2, HPAD, HW), jnp.float32),
    mesh=_mesh,
    compiler_params=pltpu.CompilerParams(use_tc_tiling_on_sc=False,
                                         needs_layout_passes=False),
    scratch_types=[
        pltpu.VMEM((AK, CH), jnp.int32),
        pltpu.VMEM((AK, CH), jnp.int32),
        pltpu.VMEM((AK, CH), jnp.int32),
        pltpu.VMEM((AK, CH), jnp.int32),
        pltpu.VMEM((CH, HW), jnp.float32),
        pltpu.VMEM((CH, HW), jnp.float32),
        pltpu.VMEM((HSW, HW), jnp.float32),
        pltpu.SemaphoreType.DMA,
        pltpu.SemaphoreType.DMA,
        pltpu.VMEM_SHARED((HPAD, HW), jnp.float32),
    ],
)
def _deg_kernel(src_hbm, dst_hbm, onl_hbm, onr_hbm, zcol_hbm, deg_hbm,
                srcb0, dstb0, srcb1, dstb1, onlb, onrb, slabb,
                d0, d1, hist):
    c = lax.axis_index("c")
    s = lax.axis_index("s")
    w = s * 2 + c
    srcb = [srcb0, srcb1]
    dstb = [dstb0, dstb1]
    dsem = [d0, d1]

    pltpu.sync_copy(zcol_hbm, slabb)
    for k in range(16):
        pltpu.sync_copy(slabb, hist.at[pl.ds(s * HSL + k * HSW, HSW)])
    pltpu.sync_copy(onl_hbm, onlb)
    pltpu.sync_copy(onr_hbm, onrb)
    plsc.subcore_barrier()

    def drain(p, n):
        def dbody(i, cc):
            pltpu.make_async_copy(onl_hbm, onlb, dsem[p]).wait()
            return cc
        lax.fori_loop(0, n, dbody, 0)

    def pair(t, carry):
        for p in range(2):
            g = 2 * t + p

            @pl.when(t > 0)
            def _():
                drain(p, 2 * AK)

            r0 = w * A_ROWS + g * AK
            pltpu.sync_copy(src_hbm.at[pl.ds(r0, AK)], srcb[p])
            pltpu.sync_copy(dst_hbm.at[pl.ds(r0, AK)], dstb[p])

            def body(j, cc):
                pltpu.async_copy(onlb, hist.at[srcb[p].at[j]],
                                 dsem[p], add=True)
                pltpu.async_copy(onrb, hist.at[dstb[p].at[j]],
                                 dsem[p], add=True)
                return cc

            lax.fori_loop(0, AK, body, 0)
        return carry

    lax.fori_loop(0, AP, pair, 0)
    for p in range(2):
        drain(p, 2 * AK)

    @pl.when(w < A_TAIL)
    def _():
        pltpu.sync_copy(src_hbm.at[pl.ds(32 * A_ROWS + w, 1)],
                        srcb0.at[pl.ds(0, 1)])
        pltpu.sync_copy(dst_hbm.at[pl.ds(32 * A_ROWS + w, 1)],
                        dstb0.at[pl.ds(0, 1)])
        pltpu.sync_copy(onlb, hist.at[srcb0.at[0]], add=True)
        pltpu.sync_copy(onrb, hist.at[dstb0.at[0]], add=True)

    plsc.subcore_barrier()

    for k in range(16):
        r = s * HSL + k * HSW
        pltpu.sync_copy(hist.at[pl.ds(r, HSW)], slabb)
        pltpu.sync_copy(slabb, deg_hbm.at[c, pl.ds(r, HSW)])


@functools.partial(
    pl.kernel,
    out_type=jax.ShapeDtypeStruct((N, DIM), jnp.float32),
    mesh=_mesh,
    compiler_params=pltpu.CompilerParams(use_tc_tiling_on_sc=False,
                                         needs_layout_passes=False),
    scratch_types=(
        [pltpu.VMEM((CK, CH), jnp.int32)] * 2
        + [pltpu.VMEM((CCAP,), jnp.int32)] * 2
        + [pltpu.VMEM((4, CH), jnp.int32)]
        + [pltpu.VMEM((CH, DIM), jnp.float32)] * 4
        + [pltpu.SemaphoreType.DMA] * 8
        + [pltpu.VMEM_SHARED((APAD, DIM), jnp.float32)]
    ),
)
def _agg_kernel(h_hbm, src_hbm, dst_hbm, zrow_hbm, out_hbm,
                srcb, dstb, csrc, cdst, idx2d, r0b, r1b, r2b, r3b,
                g0, g1, g2, g3, s0s, s1s, s2s, s3s, acc):
    c = lax.axis_index("c")
    s = lax.axis_index("s")
    base = c * HALF
    rows = [r0b, r1b, r2b, r3b]
    gsem = [g0, g1, g2, g3]
    ssem = [s0s, s1s, s2s, s3s]

    pltpu.sync_copy(zrow_hbm, r0b)
    for k in range(25):
        ch = s + 16 * k

        @pl.when(ch < NWCH)
        def _():
            pltpu.sync_copy(r0b, acc.at[pl.ds(ch * CH, CH)])

    plsc.subcore_barrier()

    def retire(q):
        pltpu.make_async_copy(h_hbm.at[idx2d.at[q]], rows[q],
                              gsem[q]).wait()
        pltpu.async_copy(rows[q], acc.at[idx2d.at[q]], ssem[q], add=True)

    def fire(cond, carry):
        n, fired, ost, tot, so0, so1, so2, so3 = carry
        sos = [so0, so1, so2, so3]
        for b in range(4):
            @pl.when(cond & (tot % 4 == b))
            def _():
                q = (b + 1) % 4

                @pl.when(ost >= 3)
                def _():
                    retire(q)

                @pl.when(sos[b] == 1)
                def _():
                    pltpu.make_async_copy(
                        rows[b], acc.at[idx2d.at[b]], ssem[b]).wait()

                def mvi(i, cc):
                    idx2d[b, pl.ds(i * 16, 16)] = (
                        cdst[pl.ds(fired * 128 + i * 16, 16)])
                    return cc
                lax.fori_loop(0, 8, mvi, 0)
                pltpu.async_copy(h_hbm.at[csrc.at[pl.ds(fired * 128, 128)]],
                                 rows[b], gsem[b])

        p = tot % 4
        qd = (tot - 3) % 4
        rearm = cond & (ost >= 3)
        so0 = jnp.where(rearm & (qd == 0), 1, so0)
        so1 = jnp.where(rearm & (qd == 1), 1, so1)
        so2 = jnp.where(rearm & (qd == 2), 1, so2)
        so3 = jnp.where(rearm & (qd == 3), 1, so3)
        so0 = jnp.where(cond & (p == 0), 0, so0)
        so1 = jnp.where(cond & (p == 1), 0, so1)
        so2 = jnp.where(cond & (p == 2), 0, so2)
        so3 = jnp.where(cond & (p == 3), 0, so3)
        fired = jnp.where(cond, fired + 1, fired)
        tot = jnp.where(cond, tot + 1, tot)
        ost = jnp.where(cond, jnp.minimum(ost + 1, 3), ost)
        return (n, fired, ost, tot, so0, so1, so2, so3)

    def flush(carry):
        n, fired, ost, tot, so0, so1, so2, so3 = carry
        for age in (3, 2, 1):
            for q in range(4):
                @pl.when((ost >= age) & ((tot - age) % 4 == q))
                def _():
                    retire(q)

            qa = (tot - age) % 4
            so0 = jnp.where((ost >= age) & (qa == 0), 1, so0)
            so1 = jnp.where((ost >= age) & (qa == 1), 1, so1)
            so2 = jnp.where((ost >= age) & (qa == 2), 1, so2)
            so3 = jnp.where((ost >= age) & (qa == 3), 1, so3)
        return (n, fired, jnp.int32(0) * ost, tot, so0, so1, so2, so3)

    def row(j, carry, enable):
        n = carry[0]

        def cvec(i, nn):
            d = dstb[j, pl.ds(i * 16, 16)]
            sv = srcb[j, pl.ds(i * 16, 16)]
            loc = d - base
            ok = (loc >= 0) & (loc < HALF) & enable
            oki = jnp.where(ok, 1, 0).astype(jnp.int32)
            cs = plsc.cumsum(oki)
            pos = jnp.where(ok, nn + cs - oki,
                            CDUMP + lax.iota(jnp.int32, 16))
            plsc.store_scatter(cdst, [pos], loc)
            plsc.store_scatter(csrc, [pos], sv)
            return nn + plsc.all_reduce_population_count(ok)

        n = lax.fori_loop(0, CH // 16, cvec, n)
        carry = (n,) + carry[1:]
        return fire(jnp.any(n >= (carry[1] + 1) * 128), carry)

    def group(g, carry):
        r0 = s * C_ROWS + g * CK
        pltpu.sync_copy(src_hbm.at[pl.ds(r0, CK)], srcb)
        pltpu.sync_copy(dst_hbm.at[pl.ds(r0, CK)], dstb)

        def rbody(j, cc):
            return row(j, cc, jnp.bool_(True))

        carry = lax.fori_loop(0, CK, rbody, carry)
        carry = flush(carry)
        n, fired = carry[0], carry[1]

        @pl.when(fired > 0)
        def _():
            def mv(i, cc):
                csrc[pl.ds(i * 16, 16)] = csrc[pl.ds(fired * 128 + i * 16, 16)]
                cdst[pl.ds(i * 16, 16)] = cdst[pl.ds(fired * 128 + i * 16, 16)]
                return cc
            lax.fori_loop(0, 8, mv, 0)

        return (n - fired * 128, jnp.int32(0) * fired) + carry[2:]

    carry = (jnp.zeros((16,), jnp.int32), jnp.int32(0), jnp.int32(0),
             jnp.int32(0), jnp.int32(0), jnp.int32(0), jnp.int32(0),
             jnp.int32(0))
    carry = lax.fori_loop(0, CG, group, carry)

    pltpu.sync_copy(src_hbm.at[pl.ds(16 * C_ROWS + s, 1)],
                    srcb.at[pl.ds(0, 1)])
    pltpu.sync_copy(dst_hbm.at[pl.ds(16 * C_ROWS + s, 1)],
                    dstb.at[pl.ds(0, 1)])
    carry = row(0, carry, jnp.bool_(True))

    @pl.when(s < 4)
    def _():
        pltpu.sync_copy(src_hbm.at[pl.ds(16 * C_ROWS + 16 + s, 1)],
                        srcb.at[pl.ds(0, 1)])
        pltpu.sync_copy(dst_hbm.at[pl.ds(16 * C_ROWS + 16 + s, 1)],
                        dstb.at[pl.ds(0, 1)])

    carry = row(0, carry, s < 4)
    carry = flush(carry)
    n, fired = carry[0], carry[1]

    for k in range(8):
        idxv = n + 16 * k + lax.iota(jnp.int32, 16)
        plsc.store_scatter(cdst, [idxv],
                           TRASH + lax.iota(jnp.int32, 16) + 16 * (k % 3))
        plsc.store_scatter(csrc, [idxv], jnp.zeros((16,), jnp.int32))

    carry = fire(jnp.any(n - fired * 128 > 0), carry)
    carry = flush(carry)
    sos = [carry[4], carry[5], carry[6], carry[7]]
    for q in range(4):
        @pl.when(sos[q] == 1)
        def _():
            pltpu.make_async_copy(rows[q], acc.at[idx2d.at[q]],
                                  ssem[q]).wait()

    plsc.subcore_barrier()

    for k in range(25):
        ch = s + 16 * k

        @pl.when(ch < NWCH - 1)
        def _():
            pltpu.sync_copy(acc.at[pl.ds(ch * CH, CH)], r0b)
            pltpu.sync_copy(r0b, out_hbm.at[pl.ds(base + ch * CH, CH)])

        @pl.when(ch == NWCH - 1)
        def _():
            pltpu.sync_copy(acc.at[pl.ds(ch * CH, 80)],
                            r0b.at[pl.ds(0, 80)])
            pltpu.sync_copy(r0b.at[pl.ds(0, 80)],
                            out_hbm.at[pl.ds(base + ch * CH, 80)])


def _norm_body(x_ref, deg_ref, h_ref, nd_ref):
    od = deg_ref[0, :, 0:1] + deg_ref[1, :, 0:1]
    idg = deg_ref[0, :, 4:5] + deg_ref[1, :, 4:5]
    ns = jnp.where(od > 0, lax.rsqrt(jnp.maximum(od, 1.0)), 0.0)
    nd = jnp.where(idg > 0, lax.rsqrt(jnp.maximum(idg, 1.0)), 0.0)
    h_ref[...] = x_ref[...] * ns
    nd_ref[...] = nd


_norm_call = pl.pallas_call(
    _norm_body,
    grid=(16,),
    in_specs=[
        pl.BlockSpec((RB, DIM), lambda i: (i, 0)),
        pl.BlockSpec((2, RB, HW), lambda i: (0, i, 0)),
    ],
    out_specs=(
        pl.BlockSpec((RB, DIM), lambda i: (i, 0)),
        pl.BlockSpec((RB, 1), lambda i: (i, 0)),
    ),
    out_shape=(
        jax.ShapeDtypeStruct((HPAD, DIM), jnp.float32),
        jax.ShapeDtypeStruct((HPAD, 1), jnp.float32),
    ),
)


def _proj_body(agg_ref, nd_ref, w_ref, b_ref, o_ref):
    a = agg_ref[...] * nd_ref[...]
    o_ref[...] = jnp.dot(a, w_ref[...],
                         preferred_element_type=jnp.float32) + b_ref[...]


_proj_call = pl.pallas_call(
    _proj_body,
    grid=(16,),
    in_specs=[
        pl.BlockSpec((RB, DIM), lambda i: (i, 0)),
        pl.BlockSpec((RB, 1), lambda i: (i, 0)),
        pl.BlockSpec((DIM, DIM), lambda i: (0, 0)),
        pl.BlockSpec((1, DIM), lambda i: (0, 0)),
    ],
    out_specs=pl.BlockSpec((RB, DIM), lambda i: (i, 0)),
    out_shape=jax.ShapeDtypeStruct((N, DIM), jnp.float32),
)


def kernel(x, edge_index, W, b):
    src2 = edge_index[0].astype(jnp.int32).reshape(ROWS, CH)
    dst2 = edge_index[1].astype(jnp.int32).reshape(ROWS, CH)
    col = jnp.arange(HW)
    ones_l = jnp.where(col < 4, 1.0, 0.0).astype(jnp.float32)
    ones_l = jnp.broadcast_to(ones_l, (CH, HW))
    ones_r = jnp.where(col >= 4, 1.0, 0.0).astype(jnp.float32)
    ones_r = jnp.broadcast_to(ones_r, (CH, HW))
    zcol = jnp.zeros((HSW, HW), jnp.float32)
    zrow = jnp.zeros((CH, DIM), jnp.float32)

    deg = _deg_kernel(src2, dst2, ones_l, ones_r, zcol)
    h, nd = _norm_call(x, deg)
    agg = _agg_kernel(h, src2, dst2, zrow)
    out = _proj_call(agg, nd, W, b.reshape(1, DIM))
    return out
